# Initial kernel scaffold; baseline (speedup 1.0000x reference)
import jax
import jax.numpy as jnp
from jax.experimental import pallas as pl


def kernel(x, masked_output, W, b, Wv, bv):
    raise NotImplementedError("write your pallas kernel here")


# speedup vs baseline: 3.6985x; 3.6985x over previous
import functools

import jax
import jax.numpy as jnp
from jax import lax
from jax.experimental import pallas as pl

LATENT = 2048
V = 2048
TR = 512


def _loss_body(xc_ref, tg_ref, w_ref, b_ref, m_ref, out_ref, *, n_rows):
    xb = xc_ref[...]
    w = w_ref[...]
    logits = (xb[:, 0:1] * w[0:1, :]
              + xb[:, 1:2] * w[1:2, :]
              + xb[:, 2:3] * w[2:3, :]) + b_ref[...]
    ml = logits * m_ref[...]
    rowmax = jnp.max(ml, axis=1, keepdims=True)
    ssum = jnp.sum(jnp.exp(ml - rowmax), axis=1, keepdims=True)
    lse = rowmax + jnp.log(ssum)
    t = tg_ref[...]
    iota = lax.broadcasted_iota(jnp.int32, ml.shape, 1)
    tl = jnp.sum(jnp.where(iota == t, ml, 0.0), axis=1, keepdims=True)
    partial = jnp.sum(lse - tl, axis=0, keepdims=True)

    step = pl.program_id(0)

    @pl.when(step == 0)
    def _init():
        out_ref[...] = jnp.zeros_like(out_ref)

    out_ref[...] += partial

    @pl.when(step == pl.num_programs(0) - 1)
    def _fin():
        out_ref[...] *= 1.0 / n_rows


def kernel(x, masked_output, W, b, Wv, bv):
    del Wv, bv
    B = x.shape[0]
    R = B * LATENT
    xc = x[:, LATENT:-1, :].reshape(R, 3)
    tg = x[:, LATENT + 1:, 0].reshape(R, 1).astype(jnp.int32)
    m2 = masked_output.reshape(R, V)
    b2 = b.reshape(1, V)

    body = functools.partial(_loss_body, n_rows=R)
    out = pl.pallas_call(
        body,
        grid=(R // TR,),
        in_specs=[
            pl.BlockSpec((TR, 3), lambda i: (i, 0)),
            pl.BlockSpec((TR, 1), lambda i: (i, 0)),
            pl.BlockSpec((3, V), lambda i: (0, 0)),
            pl.BlockSpec((1, V), lambda i: (0, 0)),
            pl.BlockSpec((TR, V), lambda i: (i, 0)),
        ],
        out_specs=pl.BlockSpec((1, 1), lambda i: (0, 0)),
        out_shape=jax.ShapeDtypeStruct((1, 1), jnp.float32),
    )(xc, tg, W, b2, m2)
    return out[0, 0]


# speedup vs baseline: 4.5432x; 1.2284x over previous
import functools

import jax
import jax.numpy as jnp
from jax import lax
from jax.experimental import pallas as pl

LATENT = 2048
V = 2048
TR = 512


def _loss_body(xc_ref, tg_ref, w_ref, m_ref, out_ref, *, n_rows):
    xb = xc_ref[...]
    w = w_ref[...]
    logits = jax.lax.dot_general(
        xb, w, (((1,), (0,)), ((), ())),
        preferred_element_type=jnp.float32)
    ml = logits * m_ref[...]
    rowmax = jnp.max(ml, axis=1, keepdims=True)
    ssum = jnp.sum(jnp.exp(ml - rowmax), axis=1, keepdims=True)
    lse = rowmax + jnp.log(ssum)
    t = tg_ref[...]
    iota = lax.broadcasted_iota(jnp.int32, ml.shape, 1)
    tl = jnp.sum(jnp.where(iota == t, ml, 0.0), axis=1, keepdims=True)
    partial = jnp.sum(lse - tl, axis=0, keepdims=True)

    step = pl.program_id(0)

    @pl.when(step == 0)
    def _init():
        out_ref[...] = jnp.zeros_like(out_ref)

    out_ref[...] += partial

    @pl.when(step == pl.num_programs(0) - 1)
    def _fin():
        out_ref[...] *= 1.0 / n_rows


def kernel(x, masked_output, W, b, Wv, bv):
    del Wv, bv
    B = x.shape[0]
    R = B * LATENT
    xc = x[:, LATENT:-1, :].reshape(R, 3)
    xa = jnp.concatenate(
        [xc, jnp.ones((R, 1), jnp.float32), jnp.zeros((R, 4), jnp.float32)],
        axis=1)
    wa = jnp.concatenate(
        [W, b.reshape(1, V), jnp.zeros((4, V), jnp.float32)], axis=0)
    tg = x[:, LATENT + 1:, 0].reshape(R, 1).astype(jnp.int32)
    m2 = masked_output.reshape(R, V)

    body = functools.partial(_loss_body, n_rows=R)
    out = pl.pallas_call(
        body,
        grid=(R // TR,),
        in_specs=[
            pl.BlockSpec((TR, 8), lambda i: (i, 0)),
            pl.BlockSpec((TR, 1), lambda i: (i, 0)),
            pl.BlockSpec((8, V), lambda i: (0, 0)),
            pl.BlockSpec((TR, V), lambda i: (i, 0)),
        ],
        out_specs=pl.BlockSpec((1, 1), lambda i: (0, 0)),
        out_shape=jax.ShapeDtypeStruct((1, 1), jnp.float32),
    )(xa, tg, wa, m2)
    return out[0, 0]
